# baseline (speedup 1.0000x reference)
import functools

import jax
import jax.numpy as jnp
from jax import lax
from jax.experimental import pallas as pl
from jax.experimental.pallas import tpu as pltpu
from jax.experimental.pallas import tpu_sc as plsc

B, NI, NT, H, D = 16, 1024, 77, 768, 64
NTP = 80
NUM_QUERY = 16
L = 16
NCH = NI // L
_SQRT_HALF = 0.7071067811865476


def _gelu(x):
    return 0.5 * x * (1.0 + lax.erf(x * _SQRT_HALF))


def _ln(x, scale, bias):
    m = x.mean(-1, keepdims=True)
    v = ((x - m) ** 2).mean(-1, keepdims=True)
    inv = 1.0 / jnp.sqrt(v + 1e-5)
    return (x - m) * inv * scale + bias


def _dense_body(img_ref, txt_ref, gum_ref,
                ln_t_scale, ln_t_bias, W_t, b_t,
                ln_i_scale, ln_i_bias, W_i, b_i,
                W_tw1, b_tw1, W_tw2, b_tw2,
                temp_ref, lam_ref, logits_out, logits_nt_out, ls_out, idx0_out):
    lin = lax.broadcasted_iota(jnp.int32, (1, NI), 1)

    t = txt_ref[0]
    x = img_ref[0]
    g = gum_ref[0]

    tc = _gelu(_ln(t, ln_t_scale[0], ln_t_bias[0]) @ W_t[...] + b_t[0]) + t[:, :D]
    tw = _gelu(t @ W_tw1[...] + b_tw1[0]) @ W_tw2[...] + b_tw2[0]
    sg = jax.nn.sigmoid(tw)
    sg_m = sg - jnp.max(sg, axis=0, keepdims=True)
    e_t = jnp.exp(sg_m)
    w = e_t / jnp.sum(e_t, axis=0, keepdims=True)
    wt = tc * w
    wt = jnp.concatenate([wt, jnp.zeros((NTP - NT, D), jnp.float32)], axis=0)

    im = _gelu(_ln(x, ln_i_scale[0], ln_i_bias[0]) @ W_i[...] + b_i[0]) + x[:, :D]

    logits = lax.dot_general(wt, im, (((1,), (1,)), ((), ())))
    logits = logits / (jnp.abs(temp_ref[0, 0]) + 1e-6)

    row = lax.broadcasted_iota(jnp.int32, (NTP, 1), 0)
    mx = jnp.max(logits, axis=1, keepdims=True)
    e = jnp.where(row < NT, jnp.exp(logits - mx), 0.0)
    s = jnp.sum(e, axis=1, keepdims=True)
    s = jnp.where(row < NT, s, 1.0)
    sm = e / s
    scores = jnp.max(sm, axis=0, keepdims=True)

    ssum = jnp.sum(scores)
    lp = jnp.log(scores / (ssum + 1e-6))
    ls = jnp.log(scores)

    u = lp + g
    m = jnp.max(u)
    idx0 = jnp.min(jnp.where(u == m, lin, NI))

    logits_out[0] = logits
    logits_nt_out[0] = lax.transpose(logits, (1, 0))
    ls_out[0] = ls
    idx0_out[0] = jnp.full((1, NUM_QUERY), idx0, jnp.int32)


@jax.jit
def _dense(image_features, text_features, gumbel,
           ln_t_scale, ln_t_bias, W_t, b_t, ln_i_scale, ln_i_bias, W_i, b_i,
           W_tw1, b_tw1, W_tw2, b_tw2, temperature, diversity_lambda):
    full = lambda shape: pl.BlockSpec(shape, lambda b: (0,) * len(shape))
    grid_spec = pl.GridSpec(
        grid=(B,),
        in_specs=[
            pl.BlockSpec((1, NI, H), lambda b: (b, 0, 0)),
            pl.BlockSpec((1, NT, H), lambda b: (b, 0, 0)),
            pl.BlockSpec((1, 1, NI), lambda b: (b, 0, 0)),
            full((1, H)), full((1, H)), full((H, D)), full((1, D)),
            full((1, H)), full((1, H)), full((H, D)), full((1, D)),
            full((H, D)), full((1, D)), full((D, 1)), full((1, 1)),
            pl.BlockSpec(memory_space=pltpu.SMEM),
            pl.BlockSpec(memory_space=pltpu.SMEM),
        ],
        out_specs=[
            pl.BlockSpec((1, NTP, NI), lambda b: (b, 0, 0)),
            pl.BlockSpec((1, NI, NTP), lambda b: (b, 0, 0)),
            pl.BlockSpec((1, 1, NI), lambda b: (b, 0, 0)),
            pl.BlockSpec((1, 1, NUM_QUERY), lambda b: (b, 0, 0)),
        ],
    )
    return pl.pallas_call(
        _dense_body,
        grid_spec=grid_spec,
        out_shape=[
            jax.ShapeDtypeStruct((B, NTP, NI), jnp.float32),
            jax.ShapeDtypeStruct((B, NI, NTP), jnp.float32),
            jax.ShapeDtypeStruct((B, 1, NI), jnp.float32),
            jax.ShapeDtypeStruct((B, 1, NUM_QUERY), jnp.int32),
        ],
    )(image_features, text_features, gumbel,
      ln_t_scale.reshape(1, H), ln_t_bias.reshape(1, H), W_t, b_t.reshape(1, D),
      ln_i_scale.reshape(1, H), ln_i_bias.reshape(1, H), W_i, b_i.reshape(1, D),
      W_tw1, b_tw1.reshape(1, D), W_tw2, b_tw2.reshape(1, 1),
      temperature.reshape(1, 1), diversity_lambda.reshape(1, 1))


def _nsqrt(x):
    xi = lax.bitcast_convert_type(x, jnp.int32)
    yi = jnp.full((L,), 0x5F3759DF, jnp.int32) - (xi >> 1)
    y = lax.bitcast_convert_type(yi, jnp.float32)
    for _ in range(5):
        y = y * (1.5 - 0.5 * x * y * y)
    return jnp.where(x > 0.0, x * y, 0.0)


def _sc_body(logits_hbm, logits_nt_hbm, ls_hbm, idx0_hbm, lam_hbm, sel_hbm,
             lg_v, ls_v, S_v, M_v, r_v, lam_v, acc_v, io_v, sel_v):
    cid = lax.axis_index("c")
    sid = lax.axis_index("s")
    j = sid * 2 + cid

    @pl.when(j < B)
    def _work():
        pltpu.sync_copy(logits_hbm.at[j], lg_v)
        pltpu.sync_copy(ls_hbm.at[j], ls_v)
        pltpu.sync_copy(idx0_hbm.at[j], io_v)
        pltpu.sync_copy(lam_hbm, lam_v)
        lam_s = lam_v[...][0]

        iota16 = lax.broadcasted_iota(jnp.int32, (L,), 0)
        zeros16 = jnp.zeros((L,), jnp.float32)
        NEG = jnp.full((L,), -jnp.inf, jnp.float32)

        cur0 = io_v[...][0]

        def zinit(c, _):
            S_v[pl.ds(c * L, L)] = zeros16
            M_v[0, pl.ds(c * L, L)] = zeros16
            return 0

        lax.fori_loop(0, NCH, zinit, 0, unroll=False)

        def round_body(k, carry):
            cur_s, sel_vec = carry
            mbase = (cur_s // L) * L
            mchunk = M_v[0, pl.ds(mbase, L)]
            M_v[0, pl.ds(mbase, L)] = jnp.where(iota16 + mbase == cur_s, NEG, mchunk)
            pltpu.sync_copy(logits_nt_hbm.at[j, pl.ds(cur_s, 1)], r_v)

            for q in range(NTP // L):
                rq = r_v[0, pl.ds(q * L, L)]
                rbs = [jnp.full((L,), rq[dd]) for dd in range(L)]

                def cbody(c, _):
                    base = c * L
                    acc = acc_v[pl.ds(base, L)] if q > 0 else zeros16
                    for dd in range(L):
                        ld = lg_v[q * L + dd, pl.ds(base, L)]
                        df = ld - rbs[dd]
                        acc = acc + df * df
                    acc_v[pl.ds(base, L)] = acc
                    return 0

                lax.fori_loop(0, NCH, cbody, 0, unroll=False)

            kf = jnp.full((L,), k.astype(jnp.float32))
            lam_vec = jnp.full((L,), lam_s)

            def merge(c, cc):
                bv, bi = cc
                base = c * L
                dist = _nsqrt(acc_v[pl.ds(base, L)])
                Sc = S_v[pl.ds(base, L)] + dist
                S_v[pl.ds(base, L)] = Sc
                comb = ls_v[pl.ds(base, L)] + lam_vec * (Sc / kf) + M_v[0, pl.ds(base, L)]
                idxs = iota16 + base
                upd = comb > bv
                bv = jnp.where(upd, comb, bv)
                bi = jnp.where(upd, idxs, bi)
                return bv, bi

            bv, bi = lax.fori_loop(0, NCH, merge,
                                   (NEG, jnp.zeros((L,), jnp.int32)), unroll=False)
            m = bv[0]
            nidx = bi[0]
            for i in range(1, L):
                bvi = bv[i]
                bii = bi[i]
                better = (bvi > m) | ((bvi == m) & (bii < nidx))
                m = jnp.where(better, bvi, m)
                nidx = jnp.where(better, bii, nidx)
            sel_vec = jnp.where(iota16 == k, jnp.full((L,), nidx), sel_vec)
            return nidx, sel_vec

        _, sel_vec = lax.fori_loop(1, NUM_QUERY, round_body,
                                   (cur0, jnp.full((L,), cur0)), unroll=False)
        sel_v[...] = sel_vec
        pltpu.sync_copy(sel_v, sel_hbm.at[j])


def _sc_select(logits, logits_nt, ls, idx0, lam16):
    mesh = plsc.VectorSubcoreMesh(core_axis_name="c", subcore_axis_name="s")
    kfn = functools.partial(
        pl.kernel, mesh=mesh,
        out_type=jax.ShapeDtypeStruct((B, NUM_QUERY), jnp.int32),
        scratch_types=[
            pltpu.VMEM((NTP, NI), jnp.float32),
            pltpu.VMEM((NI,), jnp.float32),
            pltpu.VMEM((NI,), jnp.float32),
            pltpu.VMEM((1, NI), jnp.float32),
            pltpu.VMEM((1, NTP), jnp.float32),
            pltpu.VMEM((L,), jnp.float32),
            pltpu.VMEM((NI,), jnp.float32),
            pltpu.VMEM((NUM_QUERY,), jnp.int32),
            pltpu.VMEM((NUM_QUERY,), jnp.int32),
        ],
    )(_sc_body)
    return kfn(logits, logits_nt, ls, idx0, lam16)


def kernel(image_features, text_features, ln_t_scale, ln_t_bias, W_t, b_t,
           ln_i_scale, ln_i_bias, W_i, b_i, W_tw1, b_tw1, W_tw2, b_tw2,
           temperature, diversity_lambda):
    gum = jax.random.gumbel(jax.random.key(42), (B, NI), jnp.float32)
    logits, logits_nt, ls, idx0 = _dense(
        image_features, text_features, gum.reshape(B, 1, NI),
        ln_t_scale, ln_t_bias, W_t, b_t, ln_i_scale, ln_i_bias, W_i,
        b_i, W_tw1, b_tw1, W_tw2, b_tw2,
        jnp.asarray(temperature, jnp.float32),
        jnp.asarray(diversity_lambda, jnp.float32))
    lam16 = jnp.full((L,), jnp.asarray(diversity_lambda, jnp.float32))
    sel = _sc_select(logits, logits_nt, ls.reshape(B, NI), idx0.reshape(B, NUM_QUERY), lam16)
    return sel.astype(jnp.int64)
